# Initial kernel scaffold; baseline (speedup 1.0000x reference)
import jax
import jax.numpy as jnp
from jax.experimental import pallas as pl


def kernel(x, pos, params):
    raise NotImplementedError("write your pallas kernel here")


# speedup vs baseline: 7.4339x; 7.4339x over previous
import functools

import jax
import jax.numpy as jnp
from jax import lax
from jax.experimental import pallas as pl
from jax.experimental.pallas import tpu as pltpu
from jax.experimental.pallas import tpu_sc as plsc

KNB = 32
EPS = 1e-5
NC, NS = 2, 16
NW = NC * NS


def _bf16c(v):
    return v.astype(jnp.bfloat16)


def _mmx(a, w):
    return lax.dot_general(_bf16c(a), w, (((1,), (0,)), ((), ())),
                           preferred_element_type=jnp.float32)


def _knn_body(pos_ref, post_ref, idx_ref, d_ref, *, n, rt):
    b = pl.program_id(0)
    p = pos_ref[0]
    pt = post_ref[0]
    sq_r = jnp.sum(p * p, axis=1, keepdims=True)
    sq_a = jnp.sum(pt * pt, axis=0, keepdims=True)
    acc = jnp.dot(_bf16c(p), _bf16c(pt), preferred_element_type=jnp.float32)
    d_ref[...] = sq_r - 2.0 * acc + sq_a
    cols = lax.broadcasted_iota(jnp.int32, (rt, n), 1)
    picks = []
    for _ in range(KNB):
        w = d_ref[...]
        m = jnp.min(w, axis=1, keepdims=True)
        j = jnp.min(jnp.where(w == m, cols, jnp.int32(n)), axis=1,
                    keepdims=True)
        d_ref[...] = jnp.where(cols == j, jnp.float32(3e38), w)
        picks.append(j)
    idx = jnp.concatenate(picks, axis=1)
    idx_ref[0] = idx + b * n


def _knn(pos, post, rt=256):
    bsz, n, _ = pos.shape
    return pl.pallas_call(
        functools.partial(_knn_body, n=n, rt=rt),
        grid=(bsz, n // rt),
        in_specs=[pl.BlockSpec((1, rt, 3), lambda b, t: (b, t, 0)),
                  pl.BlockSpec((1, 3, n), lambda b, t: (b, 0, 0))],
        out_specs=pl.BlockSpec((1, rt, KNB), lambda b, t: (b, t, 0)),
        out_shape=jax.ShapeDtypeStruct((bsz, n, KNB), jnp.int32),
        scratch_shapes=[pltpu.VMEM((rt, n), jnp.float32)],
    )(pos, post)


def _sc_gather(y, idxf, ch=128):
    rtot = idxf.shape[0]
    hh = y.shape[1]
    per_w = rtot // NW
    nch = per_w // ch
    mesh = plsc.VectorSubcoreMesh(core_axis_name="c", subcore_axis_name="s")

    @functools.partial(
        pl.kernel, mesh=mesh,
        out_type=jax.ShapeDtypeStruct((rtot, hh), jnp.float32),
        compiler_params=pltpu.CompilerParams(use_tc_tiling_on_sc=False),
        scratch_types=[pltpu.VMEM((per_w,), jnp.int32),
                       pltpu.VMEM((2, ch, hh), jnp.float32),
                       pltpu.SemaphoreType.DMA,
                       pltpu.SemaphoreType.DMA],
    )
    def k(y_hbm, idx_hbm, out_hbm, idx_v, rows_v, sem0, sem1):
        wid = lax.axis_index("s") * NC + lax.axis_index("c")
        base = wid * per_w
        pltpu.sync_copy(idx_hbm.at[pl.ds(base, per_w)], idx_v)
        sems = [sem0, sem1]
        cps = [None, None]
        cps[0] = pltpu.async_copy(
            y_hbm.at[idx_v.at[pl.ds(0, ch)]], rows_v.at[0], sems[0])
        for c in range(nch):
            cur = c % 2
            nxt = 1 - cur
            if c + 1 < nch:
                cps[nxt] = pltpu.async_copy(
                    y_hbm.at[idx_v.at[pl.ds((c + 1) * ch, ch)]],
                    rows_v.at[nxt], sems[nxt])
            cps[cur].wait()
            pltpu.sync_copy(rows_v.at[cur], out_hbm.at[pl.ds(base + c * ch, ch)])

    return k(y, idxf)


def _edge_h1(g_ref, xi_ref, w1_ref, b1_ref, cf):
    g = g_ref[...]
    xi = xi_ref[...]
    rn, k, cp = g.shape
    diff = g - xi[:, None, :]
    if cf < cp:
        diff = diff[:, :, :cf]
    xib = jnp.broadcast_to(xi[:, None, :cf], (rn, k, cf))
    feat = jnp.concatenate([diff, xib], axis=-1).reshape(rn * k, 2 * cf)
    return _mmx(feat, w1_ref[...]) + b1_ref[...]


def _stats1_body(g_ref, xi_ref, w1_ref, b1_ref, st_ref, *, cf):
    h1 = _edge_h1(g_ref, xi_ref, w1_ref, b1_ref, cf)
    s = jnp.sum(h1, axis=0, keepdims=True)
    q = jnp.sum(h1 * h1, axis=0, keepdims=True)

    @pl.when(pl.program_id(0) == 0)
    def _():
        st_ref[...] = jnp.zeros(st_ref.shape, st_ref.dtype)

    st_ref[0:1, :] += s
    st_ref[1:2, :] += q


def _stats1(g3, x2d, w1, b1, cf, rn=128):
    nn, k, cp = g3.shape
    hh = w1.shape[1]
    return pl.pallas_call(
        functools.partial(_stats1_body, cf=cf),
        grid=(nn // rn,),
        in_specs=[pl.BlockSpec((rn, k, cp), lambda t: (t, 0, 0)),
                  pl.BlockSpec((rn, cp), lambda t: (t, 0)),
                  pl.BlockSpec((2 * cf, hh), lambda t: (0, 0)),
                  pl.BlockSpec((1, hh), lambda t: (0, 0))],
        out_specs=pl.BlockSpec((2, hh), lambda t: (0, 0)),
        out_shape=jax.ShapeDtypeStruct((2, hh), jnp.float32),
    )(g3, x2d, w1, b1.reshape(1, hh))


def _edge_main_body(g_ref, xi_ref, w1_ref, b1_ref, st1_ref, w2_ref, b2_ref,
                    g1_ref, be1_ref, mx_ref, mn_ref, st2_ref, *, cnt, cf):
    inv = jnp.float32(1.0 / cnt)
    mean = st1_ref[0:1, :] * inv
    var = st1_ref[1:2, :] * inv - mean * mean
    a = g1_ref[...] * lax.rsqrt(var + EPS)
    c = be1_ref[...] - a * mean
    h1 = _edge_h1(g_ref, xi_ref, w1_ref, b1_ref, cf)
    hb = jnp.maximum(h1 * a + c, 0.0)
    h2 = _mmx(hb, w2_ref[...]) + b2_ref[...]
    oc = h2.shape[1]
    rn = g_ref.shape[0]
    h23 = h2.reshape(rn, KNB, oc)
    mx_ref[...] = jnp.max(h23, axis=1)
    mn_ref[...] = jnp.min(h23, axis=1)
    s = jnp.sum(h2, axis=0, keepdims=True)
    q = jnp.sum(h2 * h2, axis=0, keepdims=True)

    @pl.when(pl.program_id(0) == 0)
    def _():
        st2_ref[...] = jnp.zeros(st2_ref.shape, st2_ref.dtype)

    st2_ref[0:1, :] += s
    st2_ref[1:2, :] += q


def _edge_main(g3, x2d, w1, b1, st1, w2, b2, g1, be1, cnt, cf, rn=128):
    nn, k, cp = g3.shape
    hh = w1.shape[1]
    oc = w2.shape[1]
    return pl.pallas_call(
        functools.partial(_edge_main_body, cnt=cnt, cf=cf),
        grid=(nn // rn,),
        in_specs=[pl.BlockSpec((rn, k, cp), lambda t: (t, 0, 0)),
                  pl.BlockSpec((rn, cp), lambda t: (t, 0)),
                  pl.BlockSpec((2 * cf, hh), lambda t: (0, 0)),
                  pl.BlockSpec((1, hh), lambda t: (0, 0)),
                  pl.BlockSpec((2, hh), lambda t: (0, 0)),
                  pl.BlockSpec((hh, oc), lambda t: (0, 0)),
                  pl.BlockSpec((1, oc), lambda t: (0, 0)),
                  pl.BlockSpec((1, hh), lambda t: (0, 0)),
                  pl.BlockSpec((1, hh), lambda t: (0, 0))],
        out_specs=(pl.BlockSpec((rn, oc), lambda t: (t, 0)),
                   pl.BlockSpec((rn, oc), lambda t: (t, 0)),
                   pl.BlockSpec((2, oc), lambda t: (0, 0))),
        out_shape=(jax.ShapeDtypeStruct((nn, oc), jnp.float32),
                   jax.ShapeDtypeStruct((nn, oc), jnp.float32),
                   jax.ShapeDtypeStruct((2, oc), jnp.float32)),
    )(g3, x2d, w1, b1.reshape(1, hh), st1, w2, b2.reshape(1, oc),
      g1.reshape(1, hh), be1.reshape(1, hh))


def _finish_body(mx_ref, mn_ref, st2_ref, g2_ref, be2_ref, x_ref, *, cnt):
    inv = jnp.float32(1.0 / cnt)
    mean = st2_ref[0:1, :] * inv
    var = st2_ref[1:2, :] * inv - mean * mean
    a = g2_ref[...] * lax.rsqrt(var + EPS)
    c = be2_ref[...] - a * mean
    x_ref[...] = jnp.maximum(
        a * jnp.where(a > 0, mx_ref[...], mn_ref[...]) + c, 0.0)


def _finish(mx, mn, st2, g2, be2, cnt, rn=256):
    nn, oc = mx.shape
    return pl.pallas_call(
        functools.partial(_finish_body, cnt=cnt),
        grid=(nn // rn,),
        in_specs=[pl.BlockSpec((rn, oc), lambda t: (t, 0)),
                  pl.BlockSpec((rn, oc), lambda t: (t, 0)),
                  pl.BlockSpec((2, oc), lambda t: (0, 0)),
                  pl.BlockSpec((1, oc), lambda t: (0, 0)),
                  pl.BlockSpec((1, oc), lambda t: (0, 0))],
        out_specs=pl.BlockSpec((rn, oc), lambda t: (t, 0)),
        out_shape=jax.ShapeDtypeStruct((nn, oc), jnp.float32),
    )(mx, mn, st2, g2.reshape(1, oc), be2.reshape(1, oc))


def _hid_body(mx_ref, mn_ref, st2_ref, g2_ref, be2_ref, x1_ref, x2_ref,
              w1_ref, w2_ref, w3_ref, b_ref, st_ref, bmax_ref, bmin_ref,
              *, cnt):
    bpid = pl.program_id(0)
    t = pl.program_id(1)
    inv = jnp.float32(1.0 / cnt)
    mean = st2_ref[0:1, :] * inv
    var = st2_ref[1:2, :] * inv - mean * mean
    a = g2_ref[...] * lax.rsqrt(var + EPS)
    c = be2_ref[...] - a * mean
    x3 = jnp.maximum(a * jnp.where(a > 0, mx_ref[...], mn_ref[...]) + c, 0.0)
    h = (_mmx(x1_ref[...], w1_ref[...])
         + _mmx(x2_ref[...], w2_ref[...])
         + _mmx(x3, w3_ref[...])
         + b_ref[...])
    s = jnp.sum(h, axis=0, keepdims=True)
    q = jnp.sum(h * h, axis=0, keepdims=True)

    @pl.when(jnp.logical_and(bpid == 0, t == 0))
    def _():
        st_ref[...] = jnp.zeros(st_ref.shape, st_ref.dtype)

    st_ref[0:1, :] += s
    st_ref[1:2, :] += q

    @pl.when(t == 0)
    def _():
        bmax_ref[...] = jnp.full(bmax_ref.shape, -3e38, bmax_ref.dtype)
        bmin_ref[...] = jnp.full(bmin_ref.shape, 3e38, bmin_ref.dtype)

    bmax_ref[0] = jnp.maximum(bmax_ref[0], jnp.max(h, axis=0, keepdims=True))
    bmin_ref[0] = jnp.minimum(bmin_ref[0], jnp.min(h, axis=0, keepdims=True))


def _hid(mx, mn, st2, g2, be2, x1, x2, hid_w, hid_b, bsz, cnt2, rn=256):
    nn, oc = mx.shape
    c1 = x1.shape[1]
    c2 = x2.shape[1]
    hh = hid_w.shape[1]
    nt = nn // bsz // rn
    w1 = hid_w[:c1]
    w2 = hid_w[c1:c1 + c2]
    w3 = hid_w[c1 + c2:]
    rowmap = lambda b, t: (b * nt + t, 0)
    return pl.pallas_call(
        functools.partial(_hid_body, cnt=cnt2),
        grid=(bsz, nt),
        in_specs=[pl.BlockSpec((rn, oc), rowmap),
                  pl.BlockSpec((rn, oc), rowmap),
                  pl.BlockSpec((2, oc), lambda b, t: (0, 0)),
                  pl.BlockSpec((1, oc), lambda b, t: (0, 0)),
                  pl.BlockSpec((1, oc), lambda b, t: (0, 0)),
                  pl.BlockSpec((rn, c1), rowmap),
                  pl.BlockSpec((rn, c2), rowmap),
                  pl.BlockSpec((c1, hh), lambda b, t: (0, 0)),
                  pl.BlockSpec((c2, hh), lambda b, t: (0, 0)),
                  pl.BlockSpec((oc, hh), lambda b, t: (0, 0)),
                  pl.BlockSpec((1, hh), lambda b, t: (0, 0))],
        out_specs=(pl.BlockSpec((2, hh), lambda b, t: (0, 0)),
                   pl.BlockSpec((1, 1, hh), lambda b, t: (b, 0, 0)),
                   pl.BlockSpec((1, 1, hh), lambda b, t: (b, 0, 0))),
        out_shape=(jax.ShapeDtypeStruct((2, hh), jnp.float32),
                   jax.ShapeDtypeStruct((bsz, 1, hh), jnp.float32),
                   jax.ShapeDtypeStruct((bsz, 1, hh), jnp.float32)),
    )(mx, mn, st2, g2.reshape(1, oc), be2.reshape(1, oc), x1, x2,
      w1, w2, w3, hid_b.reshape(1, hh))


def _decoder_body(bmax_ref, bmin_ref, st_ref, hg_ref, hbe_ref, seeds_ref,
                  w11a_ref, w11b_ref, b11_ref, w12_ref, b12_ref, w13_ref,
                  b13_ref, w21a_ref, w21b_ref, b21_ref, w22_ref, b22_ref,
                  w23_ref, b23_ref, out_ref, *, cnt):
    inv = jnp.float32(1.0 / cnt)
    mean = st_ref[0:1, :] * inv
    var = st_ref[1:2, :] * inv - mean * mean
    a = hg_ref[...] * lax.rsqrt(var + EPS)
    c = hbe_ref[...] - a * mean
    code = jnp.maximum(
        a * jnp.where(a > 0, bmax_ref[0], bmin_ref[0]) + c, 0.0)
    seeds = seeds_ref[...]
    c1b = _mmx(code, w11b_ref[...]) + b11_ref[...]
    h = jnp.maximum(_mmx(seeds, w11a_ref[...]) + c1b, 0.0)
    h = jnp.maximum(_mmx(h, w12_ref[...]) + b12_ref[...], 0.0)
    fd1 = _mmx(h, w13_ref[...]) + b13_ref[...]
    c2b = _mmx(code, w21b_ref[...]) + b21_ref[...]
    h = jnp.maximum(_mmx(fd1, w21a_ref[...]) + c2b, 0.0)
    h = jnp.maximum(_mmx(h, w22_ref[...]) + b22_ref[...], 0.0)
    out_ref[0] = _mmx(h, w23_ref[...]) + b23_ref[...]


def _decoder(bmax, bmin, st, hg, hbe, seeds, p, cnt2):
    bsz, _, hh = bmax.shape
    m2 = seeds.shape[0]
    full = lambda shape: pl.BlockSpec(shape, lambda b: tuple(0 for _ in shape))
    args = (bmax, bmin, st, hg.reshape(1, hh), hbe.reshape(1, hh), seeds,
            p['f1_w1'][:2], p['f1_w1'][2:], p['f1_b1'].reshape(1, -1),
            p['f1_w2'], p['f1_b2'].reshape(1, -1),
            p['f1_w3'], p['f1_b3'].reshape(1, -1),
            p['f2_w1'][:3], p['f2_w1'][3:], p['f2_b1'].reshape(1, -1),
            p['f2_w2'], p['f2_b2'].reshape(1, -1),
            p['f2_w3'], p['f2_b3'].reshape(1, -1))
    in_specs = [pl.BlockSpec((1, 1, hh), lambda b: (b, 0, 0)),
                pl.BlockSpec((1, 1, hh), lambda b: (b, 0, 0))]
    in_specs += [full(a.shape) for a in args[2:]]
    return pl.pallas_call(
        functools.partial(_decoder_body, cnt=cnt2),
        grid=(bsz,),
        in_specs=in_specs,
        out_specs=pl.BlockSpec((1, m2, 3), lambda b: (b, 0, 0)),
        out_shape=jax.ShapeDtypeStruct((bsz, m2, 3), jnp.float32),
    )(*args)


def kernel(x, pos, params):
    bsz, n, _ = x.shape
    nn = bsz * n
    cnt1 = float(nn * KNB)
    cnt2 = float(nn)
    p = params

    post = jnp.transpose(pos, (0, 2, 1))
    idxg = _knn(pos, post)
    idxf = idxg.reshape(-1)

    x0p = jnp.pad(x.reshape(nn, 3), ((0, 0), (0, 13)))

    g3 = _sc_gather(x0p, idxf).reshape(nn, KNB, -1)
    st1 = _stats1(g3, x0p, p['c1_w1'], p['c1_b1'], 3)
    mx, mn, st2 = _edge_main(g3, x0p, p['c1_w1'], p['c1_b1'], st1,
                             p['c1_w2'], p['c1_b2'], p['c1_g1'], p['c1_be1'],
                             cnt1, 3)
    x1 = _finish(mx, mn, st2, p['c1_g2'], p['c1_be2'], cnt1)

    g3 = _sc_gather(x1, idxf).reshape(nn, KNB, -1)
    st1 = _stats1(g3, x1, p['c2_w1'], p['c2_b1'], 64)
    mx, mn, st2 = _edge_main(g3, x1, p['c2_w1'], p['c2_b1'], st1,
                             p['c2_w2'], p['c2_b2'], p['c2_g1'], p['c2_be1'],
                             cnt1, 64)
    x2 = _finish(mx, mn, st2, p['c2_g2'], p['c2_be2'], cnt1)

    g3 = _sc_gather(x2, idxf).reshape(nn, KNB, -1)
    st1 = _stats1(g3, x2, p['c3_w1'], p['c3_b1'], 128)
    mx, mn, st2 = _edge_main(g3, x2, p['c3_w1'], p['c3_b1'], st1,
                             p['c3_w2'], p['c3_b2'], p['c3_g1'], p['c3_be1'],
                             cnt1, 128)

    sth, bmax, bmin = _hid(mx, mn, st2, p['c3_g2'], p['c3_be2'], x1, x2,
                           p['hid_w'], p['hid_b'], bsz, cnt1)

    lin = jnp.linspace(-1.0, 1.0, 45)
    sa = jnp.tile(lin[None, :], (45, 1)).reshape(-1)
    sb = jnp.tile(lin[:, None], (1, 45)).reshape(-1)
    seeds = jnp.stack([sa, sb], axis=-1)
    m = seeds.shape[0]
    m2 = 2048
    seeds = jnp.pad(seeds, ((0, m2 - m), (0, 0)))

    out = _decoder(bmax, bmin, sth, p['hid_g'], p['hid_be'], seeds, p, cnt2)
    return jnp.transpose(out[:, :m, :], (0, 2, 1))
